# baseline (speedup 1.0000x reference)
import jax
import jax.numpy as jnp
from jax import lax
from jax.experimental import pallas as pl
from jax.experimental.pallas import tpu as pltpu
from jax.experimental.pallas import tpu_sc as plsc

N = 10000
E = 320000
D = 128
H = 64
DP = 144

NC = 2
NS = 16
NW = NC * NS
EPW = E // NW
K = 80
NCH = EPW // K
RPS = N // NS
PB = 15


def _leaky(x):
    return jnp.where(x > 0, x, 0.01 * x)


def _table_body(feat_ref, src_ref, rel_ref, dst_ref,
                w1, b1, w2, b2, w3, b3, out_ref, pidx_ref):
    x = feat_ref[...]
    h = _leaky(jnp.dot(x, w1[...], preferred_element_type=jnp.float32) + b1[...])
    h = _leaky(jnp.dot(h, w2[...], preferred_element_type=jnp.float32) + b2[...])
    g = jnp.dot(h, w3[...], preferred_element_type=jnp.float32) + b3[...]
    rows = out_ref.shape[1]
    pad = jnp.where(
        lax.broadcasted_iota(jnp.int32, (rows, DP - D), 1) == 0, 1.0, 0.0
    ).astype(jnp.float32)
    out_ref[0, :, 0:D] = x
    out_ref[0, :, D:DP] = pad
    out_ref[1, :, 0:D] = g
    out_ref[1, :, D:DP] = pad
    @pl.when(pl.program_id(0) == 0)
    def _pack():
        cidx = src_ref[...] + rel_ref[...] * N
        pidx_ref[...] = cidx * (2 ** PB) + dst_ref[...]


def _build_table(feat, src, rel, dst, Wi1, bi1, Wi2, bi2, Wi3, bi3):
    grid = 2
    rows = N // grid
    return pl.pallas_call(
        _table_body,
        grid=(grid,),
        in_specs=[
            pl.BlockSpec((rows, D), lambda i: (i, 0)),
            pl.BlockSpec(src.shape, lambda i: (0, 0)),
            pl.BlockSpec(src.shape, lambda i: (0, 0)),
            pl.BlockSpec(src.shape, lambda i: (0, 0)),
            pl.BlockSpec(Wi1.shape, lambda i: (0, 0)),
            pl.BlockSpec(bi1.shape, lambda i: (0, 0)),
            pl.BlockSpec(Wi2.shape, lambda i: (0, 0)),
            pl.BlockSpec(bi2.shape, lambda i: (0, 0)),
            pl.BlockSpec(Wi3.shape, lambda i: (0, 0)),
            pl.BlockSpec(bi3.shape, lambda i: (0, 0)),
        ],
        out_specs=[
            pl.BlockSpec((2, rows, DP), lambda i: (0, i, 0)),
            pl.BlockSpec(src.shape, lambda i: (0, 0)),
        ],
        out_shape=[
            jax.ShapeDtypeStruct((2, N, DP), jnp.float32),
            jax.ShapeDtypeStruct(src.shape, jnp.int32),
        ],
    )(feat, src, rel, dst, Wi1, bi1, Wi2, bi2, Wi3, bi3)


NB = 3
NT = (NCH // NB)


def _sc_body(tab, pidx, out, acc,
             p0, p1, p2, c0, c1, c2, d0, d1, d2, r0, r1, r2,
             sI0, sI1, sI2, sG0, sG1, sG2, sS0, sS1, sS2):
    c = lax.axis_index("c")
    s = lax.axis_index("s")
    w = s * NC + c
    pb = [p0, p1, p2]
    cb = [c0, c1, c2]
    db = [d0, d1, d2]
    rows = [r0, r1, r2]
    semI = [sI0, sI1, sI2]
    semG = [sG0, sG1, sG2]
    semS = [sS0, sS1, sS2]

    def _unpack(x):
        for g in range(K // 16):
            sl = pl.ds(g * 16, 16)
            p = pb[x][sl]
            cb[x][sl] = lax.shift_right_logical(p, PB)
            db[x][sl] = lax.bitwise_and(p, 2 ** PB - 1)

    def _startI(x, j):
        pltpu.make_async_copy(pidx.at[w, j], pb[x], semI[x]).start()

    def _startG(x):
        h = K // 2
        pltpu.make_async_copy(tab.at[cb[x].at[pl.ds(0, h)]],
                              rows[x].at[pl.ds(0, h)], semG[x]).start()
        pltpu.make_async_copy(tab.at[cb[x].at[pl.ds(h, h)]],
                              rows[x].at[pl.ds(h, h)], semG[x]).start()

    def _waitG(x):
        h = K // 2
        pltpu.make_async_copy(tab.at[cb[x].at[pl.ds(0, h)]],
                              rows[x].at[pl.ds(0, h)], semG[x]).wait()
        pltpu.make_async_copy(tab.at[cb[x].at[pl.ds(h, h)]],
                              rows[x].at[pl.ds(h, h)], semG[x]).wait()

    rem = RPS % K
    nzh = RPS // K + (1 if rem else 0)

    for x in range(NB):
        _startI(x, x)

    @pl.loop(0, K)
    def _zero(i):
        for j in range(DP // 16):
            r2[i, pl.ds(j * 16, 16)] = jnp.zeros((16,), jnp.float32)

    for t in range(RPS // K):
        pltpu.make_async_copy(r2, acc.at[pl.ds(s * RPS + t * K, K)],
                              sS0).start()
    if rem:
        pltpu.make_async_copy(r2.at[pl.ds(0, rem)],
                              acc.at[pl.ds(s * RPS + (RPS // K) * K, rem)],
                              sS0).start()

    for x in range(NB - 1):
        pltpu.make_async_copy(pidx.at[w, x], pb[x], semI[x]).wait()
        _unpack(x)
        _startG(x)

    for t in range(RPS // K):
        pltpu.make_async_copy(r2, acc.at[pl.ds(s * RPS + t * K, K)],
                              sS0).wait()
    if rem:
        pltpu.make_async_copy(r2.at[pl.ds(0, rem)],
                              acc.at[pl.ds(s * RPS + (RPS // K) * K, rem)],
                              sS0).wait()

    pltpu.make_async_copy(pidx.at[w, NB - 1], pb[NB - 1], semI[NB - 1]).wait()
    _unpack(NB - 1)
    _startG(NB - 1)

    plsc.subcore_barrier()

    @pl.loop(0, NT)
    def _round(t):
        j0 = t * NB
        for x in range(NB):
            _waitG(x)
            pltpu.async_copy(rows[x], acc.at[db[x]], semS[x], add=True)
            jn = j0 + x + NB

            @pl.when(jn < NCH)
            def _prefetch():
                _startI(x, jn)

        for x in range(NB):
            pltpu.make_async_copy(rows[x], acc.at[db[x]], semS[x]).wait()
            jn = j0 + x + NB

            @pl.when(jn < NCH)
            def _next():
                pltpu.make_async_copy(pidx.at[w, jn], pb[x], semI[x]).wait()
                _unpack(x)
                _startG(x)

    for x in range(NCH - NB * NT):
        _waitG(x)
        pltpu.async_copy(rows[x], acc.at[db[x]], semS[x], add=True)
    for x in range(NCH - NB * NT):
        pltpu.make_async_copy(rows[x], acc.at[db[x]], semS[x]).wait()

    plsc.subcore_barrier()

    def _wslice(h):
        n = rem if (rem and h == nzh - 1) else K
        return pl.ds(s * RPS + h * K, n), pl.ds(c * N + s * RPS + h * K, n)

    def _fill(h):
        a_sl, _ = _wslice(h)
        n = rem if (rem and h == nzh - 1) else K
        return pltpu.make_async_copy(acc.at[a_sl], rows[h % NB].at[pl.ds(0, n)],
                                     semG[h % NB])

    def _drainw(h):
        _, o_sl = _wslice(h)
        n = rem if (rem and h == nzh - 1) else K
        return pltpu.make_async_copy(rows[h % NB].at[pl.ds(0, n)], out.at[o_sl],
                                     semS[h % NB])

    for h in range(nzh):
        if h >= NB:
            _drainw(h - NB).wait()
        _fill(h).start()
        if h >= 1:
            _fill(h - 1).wait()
            _drainw(h - 1).start()
    _fill(nzh - 1).wait()
    _drainw(nzh - 1).start()
    for h in range(max(nzh - NB, 0), nzh):
        _drainw(h).wait()


def _sc_scatter(table2n, pidx3d):
    mesh = plsc.VectorSubcoreMesh(core_axis_name="c", subcore_axis_name="s")
    f = pl.kernel(
        _sc_body,
        out_type=jax.ShapeDtypeStruct((NC * N, DP), jnp.float32),
        mesh=mesh,
        scratch_types=(
            [pltpu.VMEM_SHARED((N, DP), jnp.float32)]
            + [pltpu.VMEM((K,), jnp.int32) for _ in range(3 * NB)]
            + [pltpu.VMEM((K, DP), jnp.float32) for _ in range(NB)]
            + [pltpu.SemaphoreType.DMA for _ in range(3 * NB)]
        ),
        compiler_params=pltpu.CompilerParams(use_tc_tiling_on_sc=False),
    )
    return f(table2n, pidx3d)


def _final_body(acc_ref, w1, b1, w2, b2, w3, b3, out_ref):
    sacc = acc_ref[0] + acc_ref[1]
    deg = lax.slice(sacc, (0, D), (sacc.shape[0], D + 1))
    neigh = sacc[:, 0:D] / jnp.maximum(deg, 1.0)
    h = _leaky(jnp.dot(neigh, w1[...], preferred_element_type=jnp.float32) + b1[...])
    h = _leaky(jnp.dot(h, w2[...], preferred_element_type=jnp.float32) + b2[...])
    out_ref[...] = jnp.dot(h, w3[...], preferred_element_type=jnp.float32) + b3[...]


def _finalize(acc, Wa1, ba1, Wa2, ba2, Wa3, ba3):
    grid = 2
    rows = N // grid
    return pl.pallas_call(
        _final_body,
        grid=(grid,),
        in_specs=[
            pl.BlockSpec((2, rows, DP), lambda i: (0, i, 0)),
            pl.BlockSpec(Wa1.shape, lambda i: (0, 0)),
            pl.BlockSpec(ba1.shape, lambda i: (0, 0)),
            pl.BlockSpec(Wa2.shape, lambda i: (0, 0)),
            pl.BlockSpec(ba2.shape, lambda i: (0, 0)),
            pl.BlockSpec(Wa3.shape, lambda i: (0, 0)),
            pl.BlockSpec(ba3.shape, lambda i: (0, 0)),
        ],
        out_specs=pl.BlockSpec((rows, D), lambda i: (i, 0)),
        out_shape=jax.ShapeDtypeStruct((N, D), jnp.float32),
    )(acc, Wa1, ba1, Wa2, ba2, Wa3, ba3)


def kernel(act_flag, feat, edge_index, edge_r,
           Wi1, bi1, Wi2, bi2, Wi3, bi3, Wa1, ba1, Wa2, ba2, Wa3, ba3):
    src = edge_index[0].astype(jnp.int32).reshape(E // D, D)
    rel = edge_r.astype(jnp.int32).reshape(E // D, D)
    dst = edge_index[1].astype(jnp.int32).reshape(E // D, D)

    table, pidx = _build_table(feat, src, rel, dst,
                               Wi1, bi1.reshape(1, H), Wi2, bi2.reshape(1, H),
                               Wi3, bi3.reshape(1, D))
    acc = _sc_scatter(table.reshape(2 * N, DP),
                      pidx.reshape(NW, NCH, K)).reshape(2, N, DP)
    return _finalize(acc, Wa1, ba1.reshape(1, H), Wa2, ba2.reshape(1, H),
                     Wa3, ba3.reshape(1, D))


# speedup vs baseline: 1.0027x; 1.0027x over previous
import jax
import jax.numpy as jnp
from jax import lax
from jax.experimental import pallas as pl
from jax.experimental.pallas import tpu as pltpu
from jax.experimental.pallas import tpu_sc as plsc

N = 10000
E = 320000
D = 128
H = 64
DP = 144

NC = 2
NS = 16
NW = NC * NS
EPW = E // NW
K = 80
NCH = EPW // K
RPS = N // NS
PB = 15


def _leaky(x):
    return jnp.where(x > 0, x, 0.01 * x)


def _table_body(feat_ref, src_ref, rel_ref, dst_ref,
                w1, b1, w2, b2, w3, b3, out_ref, pidx_ref):
    x = feat_ref[...]
    h = _leaky(jnp.dot(x, w1[...], preferred_element_type=jnp.float32) + b1[...])
    h = _leaky(jnp.dot(h, w2[...], preferred_element_type=jnp.float32) + b2[...])
    g = jnp.dot(h, w3[...], preferred_element_type=jnp.float32) + b3[...]
    rows = out_ref.shape[1]
    pad = jnp.where(
        lax.broadcasted_iota(jnp.int32, (rows, DP - D), 1) == 0, 1.0, 0.0
    ).astype(jnp.float32)
    out_ref[0, :, 0:D] = x
    out_ref[0, :, D:DP] = pad
    out_ref[1, :, 0:D] = g
    out_ref[1, :, D:DP] = pad
    @pl.when(pl.program_id(0) == 0)
    def _pack():
        cidx = src_ref[...] + rel_ref[...] * N
        pidx_ref[...] = cidx * (2 ** PB) + dst_ref[...]


def _build_table(feat, src, rel, dst, Wi1, bi1, Wi2, bi2, Wi3, bi3):
    grid = 5
    rows = N // grid
    return pl.pallas_call(
        _table_body,
        grid=(grid,),
        in_specs=[
            pl.BlockSpec((rows, D), lambda i: (i, 0)),
            pl.BlockSpec(src.shape, lambda i: (0, 0)),
            pl.BlockSpec(src.shape, lambda i: (0, 0)),
            pl.BlockSpec(src.shape, lambda i: (0, 0)),
            pl.BlockSpec(Wi1.shape, lambda i: (0, 0)),
            pl.BlockSpec(bi1.shape, lambda i: (0, 0)),
            pl.BlockSpec(Wi2.shape, lambda i: (0, 0)),
            pl.BlockSpec(bi2.shape, lambda i: (0, 0)),
            pl.BlockSpec(Wi3.shape, lambda i: (0, 0)),
            pl.BlockSpec(bi3.shape, lambda i: (0, 0)),
        ],
        out_specs=[
            pl.BlockSpec((2, rows, DP), lambda i: (0, i, 0)),
            pl.BlockSpec(src.shape, lambda i: (0, 0)),
        ],
        out_shape=[
            jax.ShapeDtypeStruct((2, N, DP), jnp.float32),
            jax.ShapeDtypeStruct(src.shape, jnp.int32),
        ],
    )(feat, src, rel, dst, Wi1, bi1, Wi2, bi2, Wi3, bi3)


NB = 3
NT = (NCH // NB)


def _sc_body(tab, pidx, out, acc,
             p0, p1, p2, c0, c1, c2, d0, d1, d2, r0, r1, r2,
             sI0, sI1, sI2, sG0, sG1, sG2, sS0, sS1, sS2):
    c = lax.axis_index("c")
    s = lax.axis_index("s")
    w = s * NC + c
    pb = [p0, p1, p2]
    cb = [c0, c1, c2]
    db = [d0, d1, d2]
    rows = [r0, r1, r2]
    semI = [sI0, sI1, sI2]
    semG = [sG0, sG1, sG2]
    semS = [sS0, sS1, sS2]

    def _unpack(x):
        for g in range(K // 16):
            sl = pl.ds(g * 16, 16)
            p = pb[x][sl]
            cb[x][sl] = lax.shift_right_logical(p, PB)
            db[x][sl] = lax.bitwise_and(p, 2 ** PB - 1)

    def _startI(x, j):
        pltpu.make_async_copy(pidx.at[w, j], pb[x], semI[x]).start()

    def _startG(x):
        h = K // 2
        pltpu.make_async_copy(tab.at[cb[x].at[pl.ds(0, h)]],
                              rows[x].at[pl.ds(0, h)], semG[x]).start()
        pltpu.make_async_copy(tab.at[cb[x].at[pl.ds(h, h)]],
                              rows[x].at[pl.ds(h, h)], semG[x]).start()

    def _waitG(x):
        h = K // 2
        pltpu.make_async_copy(tab.at[cb[x].at[pl.ds(0, h)]],
                              rows[x].at[pl.ds(0, h)], semG[x]).wait()
        pltpu.make_async_copy(tab.at[cb[x].at[pl.ds(h, h)]],
                              rows[x].at[pl.ds(h, h)], semG[x]).wait()

    rem = RPS % K
    nzh = RPS // K + (1 if rem else 0)

    for x in range(NB):
        _startI(x, x)

    @pl.loop(0, K)
    def _zero(i):
        for j in range(DP // 16):
            r2[i, pl.ds(j * 16, 16)] = jnp.zeros((16,), jnp.float32)

    for t in range(RPS // K):
        pltpu.make_async_copy(r2, acc.at[pl.ds(s * RPS + t * K, K)],
                              sS0).start()
    if rem:
        pltpu.make_async_copy(r2.at[pl.ds(0, rem)],
                              acc.at[pl.ds(s * RPS + (RPS // K) * K, rem)],
                              sS0).start()

    for x in range(NB - 1):
        pltpu.make_async_copy(pidx.at[w, x], pb[x], semI[x]).wait()
        _unpack(x)
        _startG(x)

    for t in range(RPS // K):
        pltpu.make_async_copy(r2, acc.at[pl.ds(s * RPS + t * K, K)],
                              sS0).wait()
    if rem:
        pltpu.make_async_copy(r2.at[pl.ds(0, rem)],
                              acc.at[pl.ds(s * RPS + (RPS // K) * K, rem)],
                              sS0).wait()

    pltpu.make_async_copy(pidx.at[w, NB - 1], pb[NB - 1], semI[NB - 1]).wait()
    _unpack(NB - 1)
    _startG(NB - 1)

    plsc.subcore_barrier()

    @pl.loop(0, NT)
    def _round(t):
        j0 = t * NB
        for x in range(NB):
            _waitG(x)
            pltpu.async_copy(rows[x], acc.at[db[x]], semS[x], add=True)
            jn = j0 + x + NB

            @pl.when(jn < NCH)
            def _prefetch():
                _startI(x, jn)

        for x in range(NB):
            pltpu.make_async_copy(rows[x], acc.at[db[x]], semS[x]).wait()
            jn = j0 + x + NB

            @pl.when(jn < NCH)
            def _next():
                pltpu.make_async_copy(pidx.at[w, jn], pb[x], semI[x]).wait()
                _unpack(x)
                _startG(x)

    for x in range(NCH - NB * NT):
        _waitG(x)
        pltpu.async_copy(rows[x], acc.at[db[x]], semS[x], add=True)
    for x in range(NCH - NB * NT):
        pltpu.make_async_copy(rows[x], acc.at[db[x]], semS[x]).wait()

    plsc.subcore_barrier()

    def _wslice(h):
        n = rem if (rem and h == nzh - 1) else K
        return pl.ds(s * RPS + h * K, n), pl.ds(c * N + s * RPS + h * K, n)

    def _fill(h):
        a_sl, _ = _wslice(h)
        n = rem if (rem and h == nzh - 1) else K
        return pltpu.make_async_copy(acc.at[a_sl], rows[h % NB].at[pl.ds(0, n)],
                                     semG[h % NB])

    def _drainw(h):
        _, o_sl = _wslice(h)
        n = rem if (rem and h == nzh - 1) else K
        return pltpu.make_async_copy(rows[h % NB].at[pl.ds(0, n)], out.at[o_sl],
                                     semS[h % NB])

    for h in range(nzh):
        if h >= NB:
            _drainw(h - NB).wait()
        _fill(h).start()
        if h >= 1:
            _fill(h - 1).wait()
            _drainw(h - 1).start()
    _fill(nzh - 1).wait()
    _drainw(nzh - 1).start()
    for h in range(max(nzh - NB, 0), nzh):
        _drainw(h).wait()


def _sc_scatter(table2n, pidx3d):
    mesh = plsc.VectorSubcoreMesh(core_axis_name="c", subcore_axis_name="s")
    f = pl.kernel(
        _sc_body,
        out_type=jax.ShapeDtypeStruct((NC * N, DP), jnp.float32),
        mesh=mesh,
        scratch_types=(
            [pltpu.VMEM_SHARED((N, DP), jnp.float32)]
            + [pltpu.VMEM((K,), jnp.int32) for _ in range(3 * NB)]
            + [pltpu.VMEM((K, DP), jnp.float32) for _ in range(NB)]
            + [pltpu.SemaphoreType.DMA for _ in range(3 * NB)]
        ),
        compiler_params=pltpu.CompilerParams(use_tc_tiling_on_sc=False),
    )
    return f(table2n, pidx3d)


def _final_body(acc_ref, w1, b1, w2, b2, w3, b3, out_ref):
    sacc = acc_ref[0] + acc_ref[1]
    deg = lax.slice(sacc, (0, D), (sacc.shape[0], D + 1))
    neigh = sacc[:, 0:D] / jnp.maximum(deg, 1.0)
    h = _leaky(jnp.dot(neigh, w1[...], preferred_element_type=jnp.float32) + b1[...])
    h = _leaky(jnp.dot(h, w2[...], preferred_element_type=jnp.float32) + b2[...])
    out_ref[...] = jnp.dot(h, w3[...], preferred_element_type=jnp.float32) + b3[...]


def _finalize(acc, Wa1, ba1, Wa2, ba2, Wa3, ba3):
    grid = 5
    rows = N // grid
    return pl.pallas_call(
        _final_body,
        grid=(grid,),
        in_specs=[
            pl.BlockSpec((2, rows, DP), lambda i: (0, i, 0)),
            pl.BlockSpec(Wa1.shape, lambda i: (0, 0)),
            pl.BlockSpec(ba1.shape, lambda i: (0, 0)),
            pl.BlockSpec(Wa2.shape, lambda i: (0, 0)),
            pl.BlockSpec(ba2.shape, lambda i: (0, 0)),
            pl.BlockSpec(Wa3.shape, lambda i: (0, 0)),
            pl.BlockSpec(ba3.shape, lambda i: (0, 0)),
        ],
        out_specs=pl.BlockSpec((rows, D), lambda i: (i, 0)),
        out_shape=jax.ShapeDtypeStruct((N, D), jnp.float32),
    )(acc, Wa1, ba1, Wa2, ba2, Wa3, ba3)


def kernel(act_flag, feat, edge_index, edge_r,
           Wi1, bi1, Wi2, bi2, Wi3, bi3, Wa1, ba1, Wa2, ba2, Wa3, ba3):
    src = edge_index[0].astype(jnp.int32).reshape(E // D, D)
    rel = edge_r.astype(jnp.int32).reshape(E // D, D)
    dst = edge_index[1].astype(jnp.int32).reshape(E // D, D)

    table, pidx = _build_table(feat, src, rel, dst,
                               Wi1, bi1.reshape(1, H), Wi2, bi2.reshape(1, H),
                               Wi3, bi3.reshape(1, D))
    acc = _sc_scatter(table.reshape(2 * N, DP),
                      pidx.reshape(NW, NCH, K)).reshape(2, N, DP)
    return _finalize(acc, Wa1, ba1.reshape(1, H), Wa2, ba2.reshape(1, H),
                     Wa3, ba3.reshape(1, D))
